# baseline (speedup 1.0000x reference)
import functools
import math

import jax
import jax.numpy as jnp
from jax import lax
from jax.experimental import pallas as pl
from jax.experimental.pallas import tpu as pltpu
from jax.experimental.pallas import tpu_sc as plsc

N = 10000
E = 160000
D = 256
H = 8
DH = 32
DFF = 1024
NUM_REL = 16
NUM_SP = 512

NP = 10240
EP = 163840
NWORK = 32
PER_W = EP // NWORK
_T0 = 3840
_T1 = 6400
_W0 = 16 * _T0

f32 = jnp.float32
i32 = jnp.int32


def _sds(shape, dtype):
    return jax.ShapeDtypeStruct(shape, dtype)


_BLKA = 256


def _qkv_body(x_ref, lnw_ref, lnb_ref, wq_ref, wk_ref, wv_ref,
              bq_ref, bk_ref, bv_ref, q_ref, k_ref, v_ref):
    x = x_ref[...]
    m = jnp.mean(x, axis=-1, keepdims=True)
    var = jnp.mean((x - m) * (x - m), axis=-1, keepdims=True)
    xn = (x - m) / jnp.sqrt(var + 1e-6) * lnw_ref[...] + lnb_ref[...]
    dn = (((1,), (1,)), ((), ()))
    q_ref[...] = lax.dot_general(xn, wq_ref[...], dn,
                                 preferred_element_type=f32) + bq_ref[...]
    k_ref[...] = lax.dot_general(xn, wk_ref[...], dn,
                                 preferred_element_type=f32) + bk_ref[...]
    v_ref[...] = lax.dot_general(xn, wv_ref[...], dn,
                                 preferred_element_type=f32) + bv_ref[...]


def _run_qkv(xp, lnw, lnb, wq, wk, wv, bq, bk, bv):
    full = pl.BlockSpec((D, D), lambda i: (0, 0))
    vec = pl.BlockSpec((1, D), lambda i: (0, 0))
    blk = pl.BlockSpec((_BLKA, D), lambda i: (i, 0))
    return pl.pallas_call(
        _qkv_body,
        grid=(NP // _BLKA,),
        in_specs=[blk, vec, vec, full, full, full, vec, vec, vec],
        out_specs=[blk, blk, blk],
        out_shape=[_sds((NP, D), f32)] * 3,
    )(xp, lnw, lnb, wq, wk, wv, bq, bk, bv)


_GC = 64


def _sc_gather_qkv_body(q_hbm, k_hbm, v_hbm, rowp_hbm, colp_hbm,
                        qe_hbm, ke_hbm, ve_hbm,
                        ra, ca, rb, cb, qa, ka, va, qb, kb, vb,
                        si, sg, so):
    c = lax.axis_index("c")
    s = lax.axis_index("s")
    base0 = jnp.where(c == 0, s * _T0, _W0 + s * _T1)
    nit = jnp.where(c == 0, _T0 // (2 * _GC), _T1 // (2 * _GC))

    def body(j, carry):
        ba = base0 + j * 2 * _GC
        bb = ba + _GC
        i0 = pltpu.async_copy(rowp_hbm.at[pl.ds(ba, _GC)], ra, si)
        i1 = pltpu.async_copy(colp_hbm.at[pl.ds(ba, _GC)], ca, si)
        i2 = pltpu.async_copy(rowp_hbm.at[pl.ds(bb, _GC)], rb, si)
        i3 = pltpu.async_copy(colp_hbm.at[pl.ds(bb, _GC)], cb, si)
        i0.wait()
        i1.wait()
        i2.wait()
        i3.wait()
        g0 = pltpu.async_copy(q_hbm.at[ca], qa, sg)
        g1 = pltpu.async_copy(k_hbm.at[ra], ka, sg)
        g2 = pltpu.async_copy(v_hbm.at[ra], va, sg)
        g3 = pltpu.async_copy(q_hbm.at[cb], qb, sg)
        g4 = pltpu.async_copy(k_hbm.at[rb], kb, sg)
        g5 = pltpu.async_copy(v_hbm.at[rb], vb, sg)
        g0.wait()
        g1.wait()
        g2.wait()
        g3.wait()
        g4.wait()
        g5.wait()
        o0 = pltpu.async_copy(qa, qe_hbm.at[pl.ds(ba, _GC)], so)
        o1 = pltpu.async_copy(ka, ke_hbm.at[pl.ds(ba, _GC)], so)
        o2 = pltpu.async_copy(va, ve_hbm.at[pl.ds(ba, _GC)], so)
        o3 = pltpu.async_copy(qb, qe_hbm.at[pl.ds(bb, _GC)], so)
        o4 = pltpu.async_copy(kb, ke_hbm.at[pl.ds(bb, _GC)], so)
        o5 = pltpu.async_copy(vb, ve_hbm.at[pl.ds(bb, _GC)], so)
        o0.wait()
        o1.wait()
        o2.wait()
        o3.wait()
        o4.wait()
        o5.wait()
        return carry

    lax.fori_loop(0, nit, body, 0)


def _run_gather_qkv(q, k, v, rowp, colp):
    mesh = plsc.VectorSubcoreMesh(core_axis_name="c", subcore_axis_name="s")
    fn = functools.partial(
        pl.kernel,
        mesh=mesh,
        out_type=[_sds((EP, D), f32)] * 3,
        scratch_types=[
            pltpu.VMEM((_GC,), i32),
            pltpu.VMEM((_GC,), i32),
            pltpu.VMEM((_GC,), i32),
            pltpu.VMEM((_GC,), i32),
            pltpu.VMEM((_GC, D), f32),
            pltpu.VMEM((_GC, D), f32),
            pltpu.VMEM((_GC, D), f32),
            pltpu.VMEM((_GC, D), f32),
            pltpu.VMEM((_GC, D), f32),
            pltpu.VMEM((_GC, D), f32),
            pltpu.SemaphoreType.DMA,
            pltpu.SemaphoreType.DMA,
            pltpu.SemaphoreType.DMA,
        ],
    )(_sc_gather_qkv_body)
    return fn(q, k, v, rowp, colp)


_BLKE = 1024


def _edge_logits_body(qe_ref, ke_ref, ve_ref, rel_ref, sp_ref, relt_ref,
                      spt_ref, ex_ref, mlo_ref, mhi_ref):
    rel_ids = rel_ref[0]
    oh_rel = (lax.broadcasted_iota(i32, (NUM_REL, _BLKE), 0)
              == rel_ids).astype(f32)
    dn0 = (((0,), (0,)), ((), ()))
    rel_e = lax.dot_general(oh_rel, relt_ref[...], dn0,
                            preferred_element_type=f32)
    qe = qe_ref[...] + rel_e
    ke = ke_ref[...] + rel_e
    prod = qe * ke
    s128 = (lax.broadcasted_iota(i32, (D, 128), 0) // DH
            == lax.broadcasted_iota(i32, (D, 128), 1)).astype(f32)
    dn1 = (((1,), (0,)), ((), ()))
    logits = lax.dot_general(prod, s128, dn1,
                             preferred_element_type=f32) / math.sqrt(DH)
    sp_ids = sp_ref[0]
    oh_sp = (lax.broadcasted_iota(i32, (NUM_SP, _BLKE), 0)
             == sp_ids).astype(f32)
    sp_e = lax.dot_general(oh_sp, spt_ref[...], dn0,
                           preferred_element_type=f32)
    head_mask = lax.broadcasted_iota(i32, (_BLKE, 128), 1) < H
    ex = jnp.where(head_mask, jnp.exp(logits + sp_e), 0.0)
    ex_ref[...] = ex
    r = (lax.broadcasted_iota(i32, (128, D), 0)
         == lax.broadcasted_iota(i32, (128, D), 1) // DH).astype(f32)
    ex256 = lax.dot_general(ex, r, dn1, preferred_element_type=f32)
    msg = ve_ref[...] * ex256
    mlo_ref[...] = msg[:, :128]
    mhi_ref[...] = msg[:, 128:]


def _run_edge_logits(qe, ke, ve, relp3, spp3, relt, spt128):
    eblk = pl.BlockSpec((_BLKE, D), lambda i: (i, 0))
    hblk = pl.BlockSpec((_BLKE, 128), lambda i: (i, 0))
    iblk = pl.BlockSpec((1, 1, _BLKE), lambda i: (i, 0, 0))
    return pl.pallas_call(
        _edge_logits_body,
        grid=(EP // _BLKE,),
        in_specs=[
            eblk, eblk, eblk, iblk, iblk,
            pl.BlockSpec((NUM_REL, D), lambda i: (0, 0)),
            pl.BlockSpec((NUM_SP, 128), lambda i: (0, 0)),
        ],
        out_specs=[hblk, hblk, hblk],
        out_shape=[_sds((EP, 128), f32)] * 3,
    )(qe, ke, ve, relp3, spp3, relt, spt128)


_BLKR = 512
_ZR = 64


def _recip_body(d0_ref, d1_ref, rw_ref):
    den = d0_ref[...] + d1_ref[...]
    lane = lax.broadcasted_iota(i32, (_BLKR, 128), 1)
    rw_ref[...] = jnp.where(lane < H, 1.0 / (den + 1e-16), 0.0)


def _run_recip(d0, d1):
    dblk = pl.BlockSpec((_BLKR, 128), lambda i: (i, 0))
    return pl.pallas_call(
        _recip_body,
        grid=(NP // _BLKR,),
        in_specs=[dblk, dblk],
        out_specs=dblk,
        out_shape=_sds((NP, 128), f32),
    )(d0, d1)


_EC = 128


def _sc_recip_gather_body(rw_hbm, colp_hbm, re_hbm,
                          ca, cb, b0, b1, si, sg, so):
    c = lax.axis_index("c")
    s = lax.axis_index("s")
    base0 = jnp.where(c == 0, s * _T0, _W0 + s * _T1)
    nit = jnp.where(c == 0, _T0 // (2 * _EC), _T1 // (2 * _EC))

    def body(j, carry):
        ba = base0 + j * 2 * _EC
        bb = ba + _EC
        i0 = pltpu.async_copy(colp_hbm.at[pl.ds(ba, _EC)], ca, si)
        i1 = pltpu.async_copy(colp_hbm.at[pl.ds(bb, _EC)], cb, si)
        i0.wait()
        i1.wait()
        g0 = pltpu.async_copy(rw_hbm.at[ca], b0, sg)
        g1 = pltpu.async_copy(rw_hbm.at[cb], b1, sg)
        g0.wait()
        g1.wait()
        o0 = pltpu.async_copy(b0, re_hbm.at[pl.ds(ba, _EC)], so)
        o1 = pltpu.async_copy(b1, re_hbm.at[pl.ds(bb, _EC)], so)
        o0.wait()
        o1.wait()
        return carry

    lax.fori_loop(0, nit, body, 0)


def _run_recip_gather(rw, colp):
    mesh = plsc.VectorSubcoreMesh(core_axis_name="c", subcore_axis_name="s")
    fn = functools.partial(
        pl.kernel,
        mesh=mesh,
        out_type=_sds((EP, 128), f32),
        scratch_types=[
            pltpu.VMEM((_EC,), i32),
            pltpu.VMEM((_EC,), i32),
            pltpu.VMEM((_EC, 128), f32),
            pltpu.VMEM((_EC, 128), f32),
            pltpu.SemaphoreType.DMA,
            pltpu.SemaphoreType.DMA,
            pltpu.SemaphoreType.DMA,
        ],
    )(_sc_recip_gather_body)
    return fn(rw, colp)


def _attn_body(ex_ref, re_ref, attn_ref):
    attn_ref[...] = (ex_ref[...] * re_ref[...])[:, :H]


def _run_attn(ex128, recip_e):
    rblk = pl.BlockSpec((_BLKE, 128), lambda i: (i, 0))
    return pl.pallas_call(
        _attn_body,
        grid=(EP // _BLKE,),
        in_specs=[rblk, rblk],
        out_specs=pl.BlockSpec((_BLKE, H), lambda i: (i, 0)),
        out_shape=_sds((EP, H), f32),
    )(ex128, recip_e)


_MC = 128


def _sc_agg_body(ex_hbm, mlo_hbm, mhi_hbm, colp_hbm,
                 d0_hbm, d1_hbm, g0l_hbm, g1l_hbm, g0h_hbm, g1h_hbm,
                 acc, zb, ma, mb, ca, cb, si, sd):
    c = lax.axis_index("c")
    s = lax.axis_index("s")
    wid = s * 2 + c

    def zbody(t, carry):
        i = t // 8
        jj = (t % 8) * 16
        zb[i, pl.ds(jj, 16)] = jnp.zeros((16,), f32)
        return carry

    lax.fori_loop(0, _ZR * 8, zbody, 0)
    rows_per_tile = NP // 16
    base0 = wid * PER_W

    def one_pass(src_hbm, o0_hbm, o1_hbm):
        def zcbody(b, carry):
            pltpu.sync_copy(
                zb, acc.at[pl.ds(s * rows_per_tile + b * _ZR, _ZR)])
            return carry

        lax.fori_loop(0, rows_per_tile // _ZR, zcbody, 0)
        plsc.subcore_barrier()

        def body(j, carry):
            ba = base0 + j * 2 * _MC
            bb = ba + _MC
            i0 = pltpu.async_copy(colp_hbm.at[pl.ds(ba, _MC)], ca, si)
            i1 = pltpu.async_copy(colp_hbm.at[pl.ds(bb, _MC)], cb, si)
            d0 = pltpu.async_copy(src_hbm.at[pl.ds(ba, _MC)], ma, sd)
            d1 = pltpu.async_copy(src_hbm.at[pl.ds(bb, _MC)], mb, sd)
            i0.wait()
            i1.wait()
            d0.wait()
            d1.wait()
            pltpu.sync_copy(ma, acc.at[ca], add=True)
            pltpu.sync_copy(mb, acc.at[cb], add=True)
            return carry

        lax.fori_loop(0, PER_W // (2 * _MC), body, 0)
        plsc.subcore_barrier()

        @pl.when(jnp.logical_and(s == 0, c == 0))
        def _():
            pltpu.sync_copy(acc, o0_hbm)

        @pl.when(jnp.logical_and(s == 0, c == 1))
        def _():
            pltpu.sync_copy(acc, o1_hbm)

        plsc.subcore_barrier()

    one_pass(ex_hbm, d0_hbm, d1_hbm)
    one_pass(mlo_hbm, g0l_hbm, g1l_hbm)
    one_pass(mhi_hbm, g0h_hbm, g1h_hbm)


def _run_agg3(ex128, msg_lo, msg_hi, colp):
    mesh = plsc.VectorSubcoreMesh(core_axis_name="c", subcore_axis_name="s")
    fn = functools.partial(
        pl.kernel,
        mesh=mesh,
        out_type=[_sds((NP, 128), f32)] * 6,
        scratch_types=[
            pltpu.VMEM_SHARED((NP, 128), f32),
            pltpu.VMEM((_ZR, 128), f32),
            pltpu.VMEM((_MC, 128), f32),
            pltpu.VMEM((_MC, 128), f32),
            pltpu.VMEM((_MC,), i32),
            pltpu.VMEM((_MC,), i32),
            pltpu.SemaphoreType.DMA,
            pltpu.SemaphoreType.DMA,
        ],
    )(_sc_agg_body)
    return fn(ex128, msg_lo, msg_hi, colp)


_BLKH = 256


def _final_body(xp_ref, g0l_ref, g1l_ref, g0h_ref, g1h_ref, rw_ref,
                dw_ref, db_ref, ln2w_ref, ln2b_ref,
                f1w_ref, f1b_ref, f2w_ref, f2b_ref, out_ref):
    agg_un = jnp.concatenate(
        [g0l_ref[...] + g1l_ref[...], g0h_ref[...] + g1h_ref[...]], axis=-1)
    r = (lax.broadcasted_iota(i32, (128, D), 0)
         == lax.broadcasted_iota(i32, (128, D), 1) // DH).astype(f32)
    rdn = (((1,), (0,)), ((), ()))
    rec256 = lax.dot_general(rw_ref[...], r, rdn, preferred_element_type=f32)
    agg = agg_un * rec256
    dn = (((1,), (1,)), ((), ()))
    attn_out = lax.dot_general(agg, dw_ref[...], dn,
                               preferred_element_type=f32) + db_ref[...]
    out1 = attn_out + xp_ref[...]
    m = jnp.mean(out1, axis=-1, keepdims=True)
    var = jnp.mean((out1 - m) * (out1 - m), axis=-1, keepdims=True)
    o1n = (out1 - m) / jnp.sqrt(var + 1e-6) * ln2w_ref[...] + ln2b_ref[...]
    h1 = jnp.maximum(
        lax.dot_general(o1n, f1w_ref[...], dn,
                        preferred_element_type=f32) + f1b_ref[...], 0.0)
    ffn = lax.dot_general(h1, f2w_ref[...], dn,
                          preferred_element_type=f32) + f2b_ref[...]
    out_ref[...] = out1 + ffn


def _run_final(xp, g0l, g1l, g0h, g1h, rw, dw, db, ln2w, ln2b,
               f1w, f1b, f2w, f2b):
    nblk = pl.BlockSpec((_BLKH, D), lambda i: (i, 0))
    hblk = pl.BlockSpec((_BLKH, 128), lambda i: (i, 0))
    vec = pl.BlockSpec((1, D), lambda i: (0, 0))
    return pl.pallas_call(
        _final_body,
        grid=(NP // _BLKH,),
        in_specs=[
            nblk, hblk, hblk, hblk, hblk, hblk,
            pl.BlockSpec((D, D), lambda i: (0, 0)), vec, vec, vec,
            pl.BlockSpec((DFF, D), lambda i: (0, 0)),
            pl.BlockSpec((1, DFF), lambda i: (0, 0)),
            pl.BlockSpec((D, DFF), lambda i: (0, 0)), vec,
        ],
        out_specs=nblk,
        out_shape=_sds((NP, D), f32),
    )(xp, g0l, g1l, g0h, g1h, rw, dw, db, ln2w, ln2b, f1w, f1b, f2w, f2b)


def kernel(feature, sp_edge_index, sp_value, edge_rel, ln1_w, ln1_b,
           wq_w, wq_b, wk_w, wk_b, wv_w, wv_b, rel_table, spatial_table,
           dense_w, dense_b, ln2_w, ln2_b, ffn1_w, ffn1_b, ffn2_w, ffn2_b):
    xp = jnp.pad(feature, ((0, NP - N), (0, 0)))
    rowp = jnp.concatenate(
        [sp_edge_index[0], jnp.full((EP - E,), N, i32)])
    colp = jnp.concatenate(
        [sp_edge_index[1], jnp.full((EP - E,), N, i32)])
    relp3 = jnp.concatenate(
        [edge_rel, jnp.zeros((EP - E,), i32)]).reshape(EP // _BLKE, 1, _BLKE)
    spp3 = jnp.concatenate(
        [sp_value, jnp.zeros((EP - E,), i32)]).reshape(EP // _BLKE, 1, _BLKE)
    spt128 = jnp.pad(spatial_table, ((0, 0), (0, 128 - H)))

    q, k, v = _run_qkv(
        xp, ln1_w.reshape(1, D), ln1_b.reshape(1, D),
        wq_w, wk_w, wv_w,
        wq_b.reshape(1, D), wk_b.reshape(1, D), wv_b.reshape(1, D))

    qe, ke, ve = _run_gather_qkv(q, k, v, rowp, colp)
    ex128, msg_lo, msg_hi = _run_edge_logits(
        qe, ke, ve, relp3, spp3, rel_table, spt128)
    d0, d1, g0l, g1l, g0h, g1h = _run_agg3(ex128, msg_lo, msg_hi, colp)
    rw = _run_recip(d0, d1)
    recip_e = _run_recip_gather(rw, colp)
    attn_pad = _run_attn(ex128, recip_e)

    out2p = _run_final(
        xp, g0l, g1l, g0h, g1h, rw, dense_w, dense_b.reshape(1, D),
        ln2_w.reshape(1, D), ln2_b.reshape(1, D),
        ffn1_w, ffn1_b.reshape(1, DFF), ffn2_w, ffn2_b.reshape(1, D))

    return (out2p[:N], attn_pad[:E])


# speedup vs baseline: 1.0240x; 1.0240x over previous
import functools
import math

import jax
import jax.numpy as jnp
from jax import lax
from jax.experimental import pallas as pl
from jax.experimental.pallas import tpu as pltpu
from jax.experimental.pallas import tpu_sc as plsc

N = 10000
E = 160000
D = 256
H = 8
DH = 32
DFF = 1024
NUM_REL = 16
NUM_SP = 512

NP = 10240
EP = 163840
NWORK = 32
PER_W = EP // NWORK

f32 = jnp.float32
i32 = jnp.int32


def _sds(shape, dtype):
    return jax.ShapeDtypeStruct(shape, dtype)


_BLKA = 256


def _qkv_body(x_ref, lnw_ref, lnb_ref, wq_ref, wk_ref, wv_ref,
              bq_ref, bk_ref, bv_ref, q_ref, k_ref, v_ref):
    x = x_ref[...]
    m = jnp.mean(x, axis=-1, keepdims=True)
    var = jnp.mean((x - m) * (x - m), axis=-1, keepdims=True)
    xn = (x - m) / jnp.sqrt(var + 1e-6) * lnw_ref[...] + lnb_ref[...]
    dn = (((1,), (1,)), ((), ()))
    q_ref[...] = lax.dot_general(xn, wq_ref[...], dn,
                                 preferred_element_type=f32) + bq_ref[...]
    k_ref[...] = lax.dot_general(xn, wk_ref[...], dn,
                                 preferred_element_type=f32) + bk_ref[...]
    v_ref[...] = lax.dot_general(xn, wv_ref[...], dn,
                                 preferred_element_type=f32) + bv_ref[...]


def _run_qkv(xp, lnw, lnb, wq, wk, wv, bq, bk, bv):
    full = pl.BlockSpec((D, D), lambda i: (0, 0))
    vec = pl.BlockSpec((1, D), lambda i: (0, 0))
    blk = pl.BlockSpec((_BLKA, D), lambda i: (i, 0))
    return pl.pallas_call(
        _qkv_body,
        grid=(NP // _BLKA,),
        in_specs=[blk, vec, vec, full, full, full, vec, vec, vec],
        out_specs=[blk, blk, blk],
        out_shape=[_sds((NP, D), f32)] * 3,
    )(xp, lnw, lnb, wq, wk, wv, bq, bk, bv)


_GC = 64


def _sc_gather_qkv_body(q_hbm, k_hbm, v_hbm, rowp_hbm, colp_hbm,
                        qe_hbm, ke_hbm, ve_hbm,
                        ra, ca, rb, cb, qa, ka, va, qb, kb, vb,
                        si, sg, so):
    c = lax.axis_index("c")
    s = lax.axis_index("s")
    base0 = (s * 2 + c) * PER_W
    nit = PER_W // (2 * _GC)

    def body(j, carry):
        ba = base0 + j * 2 * _GC
        bb = ba + _GC
        i0 = pltpu.async_copy(rowp_hbm.at[pl.ds(ba, _GC)], ra, si)
        i1 = pltpu.async_copy(colp_hbm.at[pl.ds(ba, _GC)], ca, si)
        i2 = pltpu.async_copy(rowp_hbm.at[pl.ds(bb, _GC)], rb, si)
        i3 = pltpu.async_copy(colp_hbm.at[pl.ds(bb, _GC)], cb, si)
        i0.wait()
        i1.wait()
        i2.wait()
        i3.wait()
        g0 = pltpu.async_copy(q_hbm.at[ca], qa, sg)
        g1 = pltpu.async_copy(k_hbm.at[ra], ka, sg)
        g2 = pltpu.async_copy(v_hbm.at[ra], va, sg)
        g3 = pltpu.async_copy(q_hbm.at[cb], qb, sg)
        g4 = pltpu.async_copy(k_hbm.at[rb], kb, sg)
        g5 = pltpu.async_copy(v_hbm.at[rb], vb, sg)
        g0.wait()
        g1.wait()
        g2.wait()
        g3.wait()
        g4.wait()
        g5.wait()
        o0 = pltpu.async_copy(qa, qe_hbm.at[pl.ds(ba, _GC)], so)
        o1 = pltpu.async_copy(ka, ke_hbm.at[pl.ds(ba, _GC)], so)
        o2 = pltpu.async_copy(va, ve_hbm.at[pl.ds(ba, _GC)], so)
        o3 = pltpu.async_copy(qb, qe_hbm.at[pl.ds(bb, _GC)], so)
        o4 = pltpu.async_copy(kb, ke_hbm.at[pl.ds(bb, _GC)], so)
        o5 = pltpu.async_copy(vb, ve_hbm.at[pl.ds(bb, _GC)], so)
        o0.wait()
        o1.wait()
        o2.wait()
        o3.wait()
        o4.wait()
        o5.wait()
        return carry

    lax.fori_loop(0, nit, body, 0)


def _run_gather_qkv(q, k, v, rowp, colp):
    mesh = plsc.VectorSubcoreMesh(core_axis_name="c", subcore_axis_name="s")
    fn = functools.partial(
        pl.kernel,
        mesh=mesh,
        out_type=[_sds((EP, D), f32)] * 3,
        scratch_types=[
            pltpu.VMEM((_GC,), i32),
            pltpu.VMEM((_GC,), i32),
            pltpu.VMEM((_GC,), i32),
            pltpu.VMEM((_GC,), i32),
            pltpu.VMEM((_GC, D), f32),
            pltpu.VMEM((_GC, D), f32),
            pltpu.VMEM((_GC, D), f32),
            pltpu.VMEM((_GC, D), f32),
            pltpu.VMEM((_GC, D), f32),
            pltpu.VMEM((_GC, D), f32),
            pltpu.SemaphoreType.DMA,
            pltpu.SemaphoreType.DMA,
            pltpu.SemaphoreType.DMA,
        ],
    )(_sc_gather_qkv_body)
    return fn(q, k, v, rowp, colp)


_BLKE = 1024


def _edge_logits_body(qe_ref, ke_ref, ve_ref, rel_ref, sp_ref, relt_ref,
                      spt_ref, ex_ref, mlo_ref, mhi_ref):
    rel_ids = rel_ref[0]
    oh_rel = (lax.broadcasted_iota(i32, (NUM_REL, _BLKE), 0)
              == rel_ids).astype(f32)
    dn0 = (((0,), (0,)), ((), ()))
    rel_e = lax.dot_general(oh_rel, relt_ref[...], dn0,
                            preferred_element_type=f32)
    qe = qe_ref[...] + rel_e
    ke = ke_ref[...] + rel_e
    prod = qe * ke
    s128 = (lax.broadcasted_iota(i32, (D, 128), 0) // DH
            == lax.broadcasted_iota(i32, (D, 128), 1)).astype(f32)
    dn1 = (((1,), (0,)), ((), ()))
    logits = lax.dot_general(prod, s128, dn1,
                             preferred_element_type=f32) / math.sqrt(DH)
    sp_ids = sp_ref[0]
    oh_sp = (lax.broadcasted_iota(i32, (NUM_SP, _BLKE), 0)
             == sp_ids).astype(f32)
    sp_e = lax.dot_general(oh_sp, spt_ref[...], dn0,
                           preferred_element_type=f32)
    head_mask = lax.broadcasted_iota(i32, (_BLKE, 128), 1) < H
    ex = jnp.where(head_mask, jnp.exp(logits + sp_e), 0.0)
    ex_ref[...] = ex
    r = (lax.broadcasted_iota(i32, (128, D), 0)
         == lax.broadcasted_iota(i32, (128, D), 1) // DH).astype(f32)
    ex256 = lax.dot_general(ex, r, dn1, preferred_element_type=f32)
    msg = ve_ref[...] * ex256
    mlo_ref[...] = msg[:, :128]
    mhi_ref[...] = msg[:, 128:]


def _run_edge_logits(qe, ke, ve, relp3, spp3, relt, spt128):
    eblk = pl.BlockSpec((_BLKE, D), lambda i: (i, 0))
    hblk = pl.BlockSpec((_BLKE, 128), lambda i: (i, 0))
    iblk = pl.BlockSpec((1, 1, _BLKE), lambda i: (i, 0, 0))
    return pl.pallas_call(
        _edge_logits_body,
        grid=(EP // _BLKE,),
        in_specs=[
            eblk, eblk, eblk, iblk, iblk,
            pl.BlockSpec((NUM_REL, D), lambda i: (0, 0)),
            pl.BlockSpec((NUM_SP, 128), lambda i: (0, 0)),
        ],
        out_specs=[hblk, hblk, hblk],
        out_shape=[_sds((EP, 128), f32)] * 3,
    )(qe, ke, ve, relp3, spp3, relt, spt128)


_BLKR = 512
_ZR = 64


def _recip_body(d0_ref, d1_ref, rw_ref):
    den = d0_ref[...] + d1_ref[...]
    lane = lax.broadcasted_iota(i32, (_BLKR, 128), 1)
    rw_ref[...] = jnp.where(lane < H, 1.0 / (den + 1e-16), 0.0)


def _run_recip(d0, d1):
    dblk = pl.BlockSpec((_BLKR, 128), lambda i: (i, 0))
    return pl.pallas_call(
        _recip_body,
        grid=(NP // _BLKR,),
        in_specs=[dblk, dblk],
        out_specs=dblk,
        out_shape=_sds((NP, 128), f32),
    )(d0, d1)


_EC = 128


def _sc_recip_gather_body(rw_hbm, colp_hbm, re_hbm,
                          ca, cb, b0, b1, si, sg, so):
    c = lax.axis_index("c")
    s = lax.axis_index("s")
    base0 = (s * 2 + c) * PER_W
    nit = PER_W // (2 * _EC)

    def body(j, carry):
        ba = base0 + j * 2 * _EC
        bb = ba + _EC
        i0 = pltpu.async_copy(colp_hbm.at[pl.ds(ba, _EC)], ca, si)
        i1 = pltpu.async_copy(colp_hbm.at[pl.ds(bb, _EC)], cb, si)
        i0.wait()
        i1.wait()
        g0 = pltpu.async_copy(rw_hbm.at[ca], b0, sg)
        g1 = pltpu.async_copy(rw_hbm.at[cb], b1, sg)
        g0.wait()
        g1.wait()
        o0 = pltpu.async_copy(b0, re_hbm.at[pl.ds(ba, _EC)], so)
        o1 = pltpu.async_copy(b1, re_hbm.at[pl.ds(bb, _EC)], so)
        o0.wait()
        o1.wait()
        return carry

    lax.fori_loop(0, nit, body, 0)


def _run_recip_gather(rw, colp):
    mesh = plsc.VectorSubcoreMesh(core_axis_name="c", subcore_axis_name="s")
    fn = functools.partial(
        pl.kernel,
        mesh=mesh,
        out_type=_sds((EP, 128), f32),
        scratch_types=[
            pltpu.VMEM((_EC,), i32),
            pltpu.VMEM((_EC,), i32),
            pltpu.VMEM((_EC, 128), f32),
            pltpu.VMEM((_EC, 128), f32),
            pltpu.SemaphoreType.DMA,
            pltpu.SemaphoreType.DMA,
            pltpu.SemaphoreType.DMA,
        ],
    )(_sc_recip_gather_body)
    return fn(rw, colp)


def _attn_body(ex_ref, re_ref, attn_ref):
    attn_ref[...] = (ex_ref[...] * re_ref[...])[:, :H]


def _run_attn(ex128, recip_e):
    rblk = pl.BlockSpec((_BLKE, 128), lambda i: (i, 0))
    return pl.pallas_call(
        _attn_body,
        grid=(EP // _BLKE,),
        in_specs=[rblk, rblk],
        out_specs=pl.BlockSpec((_BLKE, H), lambda i: (i, 0)),
        out_shape=_sds((EP, H), f32),
    )(ex128, recip_e)


_MC = 128


def _sc_agg_body(ex_hbm, mlo_hbm, mhi_hbm, colp_hbm,
                 d0_hbm, d1_hbm, g0l_hbm, g1l_hbm, g0h_hbm, g1h_hbm,
                 acc, zb, ma, mb, ca, cb, si, sd):
    c = lax.axis_index("c")
    s = lax.axis_index("s")
    wid = s * 2 + c

    def zbody(t, carry):
        i = t // 8
        jj = (t % 8) * 16
        zb[i, pl.ds(jj, 16)] = jnp.zeros((16,), f32)
        return carry

    lax.fori_loop(0, _ZR * 8, zbody, 0)
    rows_per_tile = NP // 16
    base0 = wid * PER_W

    def one_pass(src_hbm, o0_hbm, o1_hbm):
        def zcbody(b, carry):
            pltpu.sync_copy(
                zb, acc.at[pl.ds(s * rows_per_tile + b * _ZR, _ZR)])
            return carry

        lax.fori_loop(0, rows_per_tile // _ZR, zcbody, 0)
        plsc.subcore_barrier()

        def body(j, carry):
            ba = base0 + j * 2 * _MC
            bb = ba + _MC
            i0 = pltpu.async_copy(colp_hbm.at[pl.ds(ba, _MC)], ca, si)
            i1 = pltpu.async_copy(colp_hbm.at[pl.ds(bb, _MC)], cb, si)
            d0 = pltpu.async_copy(src_hbm.at[pl.ds(ba, _MC)], ma, sd)
            d1 = pltpu.async_copy(src_hbm.at[pl.ds(bb, _MC)], mb, sd)
            i0.wait()
            i1.wait()
            d0.wait()
            d1.wait()
            pltpu.sync_copy(ma, acc.at[ca], add=True)
            pltpu.sync_copy(mb, acc.at[cb], add=True)
            return carry

        lax.fori_loop(0, PER_W // (2 * _MC), body, 0)
        plsc.subcore_barrier()

        @pl.when(jnp.logical_and(s == 0, c == 0))
        def _():
            pltpu.sync_copy(acc, o0_hbm)

        @pl.when(jnp.logical_and(s == 0, c == 1))
        def _():
            pltpu.sync_copy(acc, o1_hbm)

        plsc.subcore_barrier()

    one_pass(ex_hbm, d0_hbm, d1_hbm)
    one_pass(mlo_hbm, g0l_hbm, g1l_hbm)
    one_pass(mhi_hbm, g0h_hbm, g1h_hbm)


def _run_agg3(ex128, msg_lo, msg_hi, colp):
    mesh = plsc.VectorSubcoreMesh(core_axis_name="c", subcore_axis_name="s")
    fn = functools.partial(
        pl.kernel,
        mesh=mesh,
        out_type=[_sds((NP, 128), f32)] * 6,
        scratch_types=[
            pltpu.VMEM_SHARED((NP, 128), f32),
            pltpu.VMEM((_ZR, 128), f32),
            pltpu.VMEM((_MC, 128), f32),
            pltpu.VMEM((_MC, 128), f32),
            pltpu.VMEM((_MC,), i32),
            pltpu.VMEM((_MC,), i32),
            pltpu.SemaphoreType.DMA,
            pltpu.SemaphoreType.DMA,
        ],
    )(_sc_agg_body)
    return fn(ex128, msg_lo, msg_hi, colp)


_BLKH = 256


def _final_body(xp_ref, g0l_ref, g1l_ref, g0h_ref, g1h_ref, rw_ref,
                dw_ref, db_ref, ln2w_ref, ln2b_ref,
                f1w_ref, f1b_ref, f2w_ref, f2b_ref, out_ref):
    agg_un = jnp.concatenate(
        [g0l_ref[...] + g1l_ref[...], g0h_ref[...] + g1h_ref[...]], axis=-1)
    r = (lax.broadcasted_iota(i32, (128, D), 0)
         == lax.broadcasted_iota(i32, (128, D), 1) // DH).astype(f32)
    rdn = (((1,), (0,)), ((), ()))
    rec256 = lax.dot_general(rw_ref[...], r, rdn, preferred_element_type=f32)
    agg = agg_un * rec256
    dn = (((1,), (1,)), ((), ()))
    attn_out = lax.dot_general(agg, dw_ref[...], dn,
                               preferred_element_type=f32) + db_ref[...]
    out1 = attn_out + xp_ref[...]
    m = jnp.mean(out1, axis=-1, keepdims=True)
    var = jnp.mean((out1 - m) * (out1 - m), axis=-1, keepdims=True)
    o1n = (out1 - m) / jnp.sqrt(var + 1e-6) * ln2w_ref[...] + ln2b_ref[...]
    h1 = jnp.maximum(
        lax.dot_general(o1n, f1w_ref[...], dn,
                        preferred_element_type=f32) + f1b_ref[...], 0.0)
    ffn = lax.dot_general(h1, f2w_ref[...], dn,
                          preferred_element_type=f32) + f2b_ref[...]
    out_ref[...] = out1 + ffn


def _run_final(xp, g0l, g1l, g0h, g1h, rw, dw, db, ln2w, ln2b,
               f1w, f1b, f2w, f2b):
    nblk = pl.BlockSpec((_BLKH, D), lambda i: (i, 0))
    hblk = pl.BlockSpec((_BLKH, 128), lambda i: (i, 0))
    vec = pl.BlockSpec((1, D), lambda i: (0, 0))
    return pl.pallas_call(
        _final_body,
        grid=(NP // _BLKH,),
        in_specs=[
            nblk, hblk, hblk, hblk, hblk, hblk,
            pl.BlockSpec((D, D), lambda i: (0, 0)), vec, vec, vec,
            pl.BlockSpec((DFF, D), lambda i: (0, 0)),
            pl.BlockSpec((1, DFF), lambda i: (0, 0)),
            pl.BlockSpec((D, DFF), lambda i: (0, 0)), vec,
        ],
        out_specs=nblk,
        out_shape=_sds((NP, D), f32),
    )(xp, g0l, g1l, g0h, g1h, rw, dw, db, ln2w, ln2b, f1w, f1b, f2w, f2b)


def kernel(feature, sp_edge_index, sp_value, edge_rel, ln1_w, ln1_b,
           wq_w, wq_b, wk_w, wk_b, wv_w, wv_b, rel_table, spatial_table,
           dense_w, dense_b, ln2_w, ln2_b, ffn1_w, ffn1_b, ffn2_w, ffn2_b):
    xp = jnp.pad(feature, ((0, NP - N), (0, 0)))
    rowp = jnp.concatenate(
        [sp_edge_index[0], jnp.full((EP - E,), N, i32)])
    colp = jnp.concatenate(
        [sp_edge_index[1], jnp.full((EP - E,), N, i32)])
    relp3 = jnp.concatenate(
        [edge_rel, jnp.zeros((EP - E,), i32)]).reshape(EP // _BLKE, 1, _BLKE)
    spp3 = jnp.concatenate(
        [sp_value, jnp.zeros((EP - E,), i32)]).reshape(EP // _BLKE, 1, _BLKE)
    spt128 = jnp.pad(spatial_table, ((0, 0), (0, 128 - H)))

    q, k, v = _run_qkv(
        xp, ln1_w.reshape(1, D), ln1_b.reshape(1, D),
        wq_w, wk_w, wv_w,
        wq_b.reshape(1, D), wk_b.reshape(1, D), wv_b.reshape(1, D))

    qe, ke, ve = _run_gather_qkv(q, k, v, rowp, colp)
    ex128, msg_lo, msg_hi = _run_edge_logits(
        qe, ke, ve, relp3, spp3, rel_table, spt128)
    d0, d1, g0l, g1l, g0h, g1h = _run_agg3(ex128, msg_lo, msg_hi, colp)
    rw = _run_recip(d0, d1)
    recip_e = _run_recip_gather(rw, colp)
    attn_pad = _run_attn(ex128, recip_e)

    out2p = _run_final(
        xp, g0l, g1l, g0h, g1h, rw, dense_w, dense_b.reshape(1, D),
        ln2_w.reshape(1, D), ln2_b.reshape(1, D),
        ffn1_w, ffn1_b.reshape(1, DFF), ffn2_w, ffn2_b.reshape(1, D))

    return (out2p[:N], attn_pad[:E])


# speedup vs baseline: 1.0306x; 1.0064x over previous
import functools
import math

import jax
import jax.numpy as jnp
from jax import lax
from jax.experimental import pallas as pl
from jax.experimental.pallas import tpu as pltpu
from jax.experimental.pallas import tpu_sc as plsc

N = 10000
E = 160000
D = 256
H = 8
DH = 32
DFF = 1024
NUM_REL = 16
NUM_SP = 512

NP = 10240
EP = 163840
NWORK = 32
PER_W = EP // NWORK

f32 = jnp.float32
i32 = jnp.int32


def _sds(shape, dtype):
    return jax.ShapeDtypeStruct(shape, dtype)


_BLKA = 256


def _qkv_body(x_ref, lnw_ref, lnb_ref, wq_ref, wk_ref, wv_ref,
              bq_ref, bk_ref, bv_ref, q_ref, k_ref, v_ref):
    x = x_ref[...]
    m = jnp.mean(x, axis=-1, keepdims=True)
    var = jnp.mean((x - m) * (x - m), axis=-1, keepdims=True)
    xn = (x - m) / jnp.sqrt(var + 1e-6) * lnw_ref[...] + lnb_ref[...]
    dn = (((1,), (1,)), ((), ()))
    q_ref[...] = lax.dot_general(xn, wq_ref[...], dn,
                                 preferred_element_type=f32) + bq_ref[...]
    k_ref[...] = lax.dot_general(xn, wk_ref[...], dn,
                                 preferred_element_type=f32) + bk_ref[...]
    v_ref[...] = lax.dot_general(xn, wv_ref[...], dn,
                                 preferred_element_type=f32) + bv_ref[...]


def _run_qkv(xp, lnw, lnb, wq, wk, wv, bq, bk, bv):
    full = pl.BlockSpec((D, D), lambda i: (0, 0))
    vec = pl.BlockSpec((1, D), lambda i: (0, 0))
    blk = pl.BlockSpec((_BLKA, D), lambda i: (i, 0))
    return pl.pallas_call(
        _qkv_body,
        grid=(NP // _BLKA,),
        in_specs=[blk, vec, vec, full, full, full, vec, vec, vec],
        out_specs=[blk, blk, blk],
        out_shape=[_sds((NP, D), f32)] * 3,
    )(xp, lnw, lnb, wq, wk, wv, bq, bk, bv)


_GC = 64


def _sc_gather_qkv_body(q_hbm, k_hbm, v_hbm, rowp_hbm, colp_hbm,
                        qe_hbm, ke_hbm, ve_hbm,
                        ra, ca, rb, cb, qa, ka, va, qb, kb, vb,
                        si, sg, so):
    c = lax.axis_index("c")
    s = lax.axis_index("s")
    base0 = (s * 2 + c) * PER_W
    nit = PER_W // (2 * _GC)

    def body(j, carry):
        ba = base0 + j * 2 * _GC
        bb = ba + _GC
        i0 = pltpu.async_copy(rowp_hbm.at[pl.ds(ba, _GC)], ra, si)
        i1 = pltpu.async_copy(colp_hbm.at[pl.ds(ba, _GC)], ca, si)
        i2 = pltpu.async_copy(rowp_hbm.at[pl.ds(bb, _GC)], rb, si)
        i3 = pltpu.async_copy(colp_hbm.at[pl.ds(bb, _GC)], cb, si)
        i0.wait()
        i1.wait()
        i2.wait()
        i3.wait()
        g0 = pltpu.async_copy(q_hbm.at[ca], qa, sg)
        g1 = pltpu.async_copy(k_hbm.at[ra], ka, sg)
        g2 = pltpu.async_copy(v_hbm.at[ra], va, sg)
        g3 = pltpu.async_copy(q_hbm.at[cb], qb, sg)
        g4 = pltpu.async_copy(k_hbm.at[rb], kb, sg)
        g5 = pltpu.async_copy(v_hbm.at[rb], vb, sg)
        g0.wait()
        g1.wait()
        g2.wait()
        g3.wait()
        g4.wait()
        g5.wait()
        o0 = pltpu.async_copy(qa, qe_hbm.at[pl.ds(ba, _GC)], so)
        o1 = pltpu.async_copy(ka, ke_hbm.at[pl.ds(ba, _GC)], so)
        o2 = pltpu.async_copy(va, ve_hbm.at[pl.ds(ba, _GC)], so)
        o3 = pltpu.async_copy(qb, qe_hbm.at[pl.ds(bb, _GC)], so)
        o4 = pltpu.async_copy(kb, ke_hbm.at[pl.ds(bb, _GC)], so)
        o5 = pltpu.async_copy(vb, ve_hbm.at[pl.ds(bb, _GC)], so)
        o0.wait()
        o1.wait()
        o2.wait()
        o3.wait()
        o4.wait()
        o5.wait()
        return carry

    lax.fori_loop(0, nit, body, 0)


def _run_gather_qkv(q, k, v, rowp, colp):
    mesh = plsc.VectorSubcoreMesh(core_axis_name="c", subcore_axis_name="s")
    fn = functools.partial(
        pl.kernel,
        mesh=mesh,
        out_type=[_sds((EP, D), f32)] * 3,
        scratch_types=[
            pltpu.VMEM((_GC,), i32),
            pltpu.VMEM((_GC,), i32),
            pltpu.VMEM((_GC,), i32),
            pltpu.VMEM((_GC,), i32),
            pltpu.VMEM((_GC, D), f32),
            pltpu.VMEM((_GC, D), f32),
            pltpu.VMEM((_GC, D), f32),
            pltpu.VMEM((_GC, D), f32),
            pltpu.VMEM((_GC, D), f32),
            pltpu.VMEM((_GC, D), f32),
            pltpu.SemaphoreType.DMA,
            pltpu.SemaphoreType.DMA,
            pltpu.SemaphoreType.DMA,
        ],
    )(_sc_gather_qkv_body)
    return fn(q, k, v, rowp, colp)


_BLKE = 1024


def _edge_logits_body(qe_ref, ke_ref, ve_ref, rel_ref, sp_ref, relt_ref,
                      spt_ref, ex_ref, mlo_ref, mhi_ref):
    rel_ids = rel_ref[0]
    oh_rel = (lax.broadcasted_iota(i32, (NUM_REL, _BLKE), 0)
              == rel_ids).astype(f32)
    dn0 = (((0,), (0,)), ((), ()))
    rel_e = lax.dot_general(oh_rel, relt_ref[...], dn0,
                            preferred_element_type=f32)
    qe = qe_ref[...] + rel_e
    ke = ke_ref[...] + rel_e
    prod = qe * ke
    s128 = (lax.broadcasted_iota(i32, (D, 128), 0) // DH
            == lax.broadcasted_iota(i32, (D, 128), 1)).astype(f32)
    dn1 = (((1,), (0,)), ((), ()))
    logits = lax.dot_general(prod, s128, dn1,
                             preferred_element_type=f32) / math.sqrt(DH)
    sp_ids = sp_ref[0]
    oh_sp = (lax.broadcasted_iota(i32, (NUM_SP, _BLKE), 0)
             == sp_ids).astype(f32)
    sp_e = lax.dot_general(oh_sp, spt_ref[...], dn0,
                           preferred_element_type=f32)
    head_mask = lax.broadcasted_iota(i32, (_BLKE, 128), 1) < H
    ex = jnp.where(head_mask, jnp.exp(logits + sp_e), 0.0)
    ex_ref[...] = ex
    r = (lax.broadcasted_iota(i32, (128, D), 0)
         == lax.broadcasted_iota(i32, (128, D), 1) // DH).astype(f32)
    ex256 = lax.dot_general(ex, r, dn1, preferred_element_type=f32)
    msg = ve_ref[...] * ex256
    mlo_ref[...] = msg[:, :128]
    mhi_ref[...] = msg[:, 128:]


def _run_edge_logits(qe, ke, ve, relp3, spp3, relt, spt128):
    eblk = pl.BlockSpec((_BLKE, D), lambda i: (i, 0))
    hblk = pl.BlockSpec((_BLKE, 128), lambda i: (i, 0))
    iblk = pl.BlockSpec((1, 1, _BLKE), lambda i: (i, 0, 0))
    return pl.pallas_call(
        _edge_logits_body,
        grid=(EP // _BLKE,),
        in_specs=[
            eblk, eblk, eblk, iblk, iblk,
            pl.BlockSpec((NUM_REL, D), lambda i: (0, 0)),
            pl.BlockSpec((NUM_SP, 128), lambda i: (0, 0)),
        ],
        out_specs=[hblk, hblk, hblk],
        out_shape=[_sds((EP, 128), f32)] * 3,
    )(qe, ke, ve, relp3, spp3, relt, spt128)


_BLKR = 512
_ZR = 64


def _recip_body(d0_ref, d1_ref, rw_ref):
    den = d0_ref[...] + d1_ref[...]
    lane = lax.broadcasted_iota(i32, (_BLKR, 128), 1)
    rw_ref[...] = jnp.where(lane < H, 1.0 / (den + 1e-16), 0.0)


def _run_recip(d0, d1):
    dblk = pl.BlockSpec((_BLKR, 128), lambda i: (i, 0))
    return pl.pallas_call(
        _recip_body,
        grid=(NP // _BLKR,),
        in_specs=[dblk, dblk],
        out_specs=dblk,
        out_shape=_sds((NP, 128), f32),
    )(d0, d1)


_EC = 128


def _sc_recip_gather_body(rw_hbm, colp_hbm, re_hbm,
                          c0, c1, c2, c3, b0, b1, b2, b3, si, sg, so):
    c = lax.axis_index("c")
    s = lax.axis_index("s")
    base0 = (s * 2 + c) * PER_W
    nit = PER_W // (4 * _EC)
    cbs = (c0, c1, c2, c3)
    bbs = (b0, b1, b2, b3)

    def body(j, carry):
        b = base0 + j * 4 * _EC
        iws = [pltpu.async_copy(colp_hbm.at[pl.ds(b + t * _EC, _EC)], cbs[t],
                                si) for t in range(4)]
        for w in iws:
            w.wait()
        gws = [pltpu.async_copy(rw_hbm.at[cbs[t]], bbs[t], sg)
               for t in range(4)]
        for w in gws:
            w.wait()
        ows = [pltpu.async_copy(bbs[t], re_hbm.at[pl.ds(b + t * _EC, _EC)],
                                so) for t in range(4)]
        for w in ows:
            w.wait()
        return carry

    lax.fori_loop(0, nit, body, 0)


def _run_recip_gather(rw, colp):
    mesh = plsc.VectorSubcoreMesh(core_axis_name="c", subcore_axis_name="s")
    fn = functools.partial(
        pl.kernel,
        mesh=mesh,
        out_type=_sds((EP, 128), f32),
        scratch_types=(
            [pltpu.VMEM((_EC,), i32)] * 4
            + [pltpu.VMEM((_EC, 128), f32)] * 4
            + [pltpu.SemaphoreType.DMA] * 3
        ),
    )(_sc_recip_gather_body)
    return fn(rw, colp)


def _attn_body(ex_ref, re_ref, attn_ref):
    attn_ref[...] = (ex_ref[...] * re_ref[...])[:, :H]


def _run_attn(ex128, recip_e):
    rblk = pl.BlockSpec((_BLKE, 128), lambda i: (i, 0))
    return pl.pallas_call(
        _attn_body,
        grid=(EP // _BLKE,),
        in_specs=[rblk, rblk],
        out_specs=pl.BlockSpec((_BLKE, H), lambda i: (i, 0)),
        out_shape=_sds((EP, H), f32),
    )(ex128, recip_e)


_MC = 128


def _sc_agg_body(ex_hbm, mlo_hbm, mhi_hbm, colp_hbm,
                 d0_hbm, d1_hbm, g0l_hbm, g1l_hbm, g0h_hbm, g1h_hbm,
                 acc, zb, ma, mb, ca, cb, si, sd):
    c = lax.axis_index("c")
    s = lax.axis_index("s")
    wid = s * 2 + c

    def zbody(t, carry):
        i = t // 8
        jj = (t % 8) * 16
        zb[i, pl.ds(jj, 16)] = jnp.zeros((16,), f32)
        return carry

    lax.fori_loop(0, _ZR * 8, zbody, 0)
    rows_per_tile = NP // 16
    base0 = wid * PER_W

    def one_pass(src_hbm, o0_hbm, o1_hbm):
        def zcbody(b, carry):
            pltpu.sync_copy(
                zb, acc.at[pl.ds(s * rows_per_tile + b * _ZR, _ZR)])
            return carry

        lax.fori_loop(0, rows_per_tile // _ZR, zcbody, 0)
        plsc.subcore_barrier()

        def body(j, carry):
            ba = base0 + j * 2 * _MC
            bb = ba + _MC
            i0 = pltpu.async_copy(colp_hbm.at[pl.ds(ba, _MC)], ca, si)
            i1 = pltpu.async_copy(colp_hbm.at[pl.ds(bb, _MC)], cb, si)
            d0 = pltpu.async_copy(src_hbm.at[pl.ds(ba, _MC)], ma, sd)
            d1 = pltpu.async_copy(src_hbm.at[pl.ds(bb, _MC)], mb, sd)
            i0.wait()
            i1.wait()
            d0.wait()
            d1.wait()
            pltpu.sync_copy(ma, acc.at[ca], add=True)
            pltpu.sync_copy(mb, acc.at[cb], add=True)
            return carry

        lax.fori_loop(0, PER_W // (2 * _MC), body, 0)
        plsc.subcore_barrier()

        @pl.when(jnp.logical_and(s == 0, c == 0))
        def _():
            pltpu.sync_copy(acc, o0_hbm)

        @pl.when(jnp.logical_and(s == 0, c == 1))
        def _():
            pltpu.sync_copy(acc, o1_hbm)

        plsc.subcore_barrier()

    one_pass(ex_hbm, d0_hbm, d1_hbm)
    one_pass(mlo_hbm, g0l_hbm, g1l_hbm)
    one_pass(mhi_hbm, g0h_hbm, g1h_hbm)


def _run_agg3(ex128, msg_lo, msg_hi, colp):
    mesh = plsc.VectorSubcoreMesh(core_axis_name="c", subcore_axis_name="s")
    fn = functools.partial(
        pl.kernel,
        mesh=mesh,
        out_type=[_sds((NP, 128), f32)] * 6,
        scratch_types=[
            pltpu.VMEM_SHARED((NP, 128), f32),
            pltpu.VMEM((_ZR, 128), f32),
            pltpu.VMEM((_MC, 128), f32),
            pltpu.VMEM((_MC, 128), f32),
            pltpu.VMEM((_MC,), i32),
            pltpu.VMEM((_MC,), i32),
            pltpu.SemaphoreType.DMA,
            pltpu.SemaphoreType.DMA,
        ],
    )(_sc_agg_body)
    return fn(ex128, msg_lo, msg_hi, colp)


_BLKH = 256


def _final_body(xp_ref, g0l_ref, g1l_ref, g0h_ref, g1h_ref, rw_ref,
                dw_ref, db_ref, ln2w_ref, ln2b_ref,
                f1w_ref, f1b_ref, f2w_ref, f2b_ref, out_ref):
    agg_un = jnp.concatenate(
        [g0l_ref[...] + g1l_ref[...], g0h_ref[...] + g1h_ref[...]], axis=-1)
    r = (lax.broadcasted_iota(i32, (128, D), 0)
         == lax.broadcasted_iota(i32, (128, D), 1) // DH).astype(f32)
    rdn = (((1,), (0,)), ((), ()))
    rec256 = lax.dot_general(rw_ref[...], r, rdn, preferred_element_type=f32)
    agg = agg_un * rec256
    dn = (((1,), (1,)), ((), ()))
    attn_out = lax.dot_general(agg, dw_ref[...], dn,
                               preferred_element_type=f32) + db_ref[...]
    out1 = attn_out + xp_ref[...]
    m = jnp.mean(out1, axis=-1, keepdims=True)
    var = jnp.mean((out1 - m) * (out1 - m), axis=-1, keepdims=True)
    o1n = (out1 - m) / jnp.sqrt(var + 1e-6) * ln2w_ref[...] + ln2b_ref[...]
    h1 = jnp.maximum(
        lax.dot_general(o1n, f1w_ref[...], dn,
                        preferred_element_type=f32) + f1b_ref[...], 0.0)
    ffn = lax.dot_general(h1, f2w_ref[...], dn,
                          preferred_element_type=f32) + f2b_ref[...]
    out_ref[...] = out1 + ffn


def _run_final(xp, g0l, g1l, g0h, g1h, rw, dw, db, ln2w, ln2b,
               f1w, f1b, f2w, f2b):
    nblk = pl.BlockSpec((_BLKH, D), lambda i: (i, 0))
    hblk = pl.BlockSpec((_BLKH, 128), lambda i: (i, 0))
    vec = pl.BlockSpec((1, D), lambda i: (0, 0))
    return pl.pallas_call(
        _final_body,
        grid=(NP // _BLKH,),
        in_specs=[
            nblk, hblk, hblk, hblk, hblk, hblk,
            pl.BlockSpec((D, D), lambda i: (0, 0)), vec, vec, vec,
            pl.BlockSpec((DFF, D), lambda i: (0, 0)),
            pl.BlockSpec((1, DFF), lambda i: (0, 0)),
            pl.BlockSpec((D, DFF), lambda i: (0, 0)), vec,
        ],
        out_specs=nblk,
        out_shape=_sds((NP, D), f32),
    )(xp, g0l, g1l, g0h, g1h, rw, dw, db, ln2w, ln2b, f1w, f1b, f2w, f2b)


def kernel(feature, sp_edge_index, sp_value, edge_rel, ln1_w, ln1_b,
           wq_w, wq_b, wk_w, wk_b, wv_w, wv_b, rel_table, spatial_table,
           dense_w, dense_b, ln2_w, ln2_b, ffn1_w, ffn1_b, ffn2_w, ffn2_b):
    xp = jnp.pad(feature, ((0, NP - N), (0, 0)))
    rowp = jnp.concatenate(
        [sp_edge_index[0], jnp.full((EP - E,), N, i32)])
    colp = jnp.concatenate(
        [sp_edge_index[1], jnp.full((EP - E,), N, i32)])
    relp3 = jnp.concatenate(
        [edge_rel, jnp.zeros((EP - E,), i32)]).reshape(EP // _BLKE, 1, _BLKE)
    spp3 = jnp.concatenate(
        [sp_value, jnp.zeros((EP - E,), i32)]).reshape(EP // _BLKE, 1, _BLKE)
    spt128 = jnp.pad(spatial_table, ((0, 0), (0, 128 - H)))

    q, k, v = _run_qkv(
        xp, ln1_w.reshape(1, D), ln1_b.reshape(1, D),
        wq_w, wk_w, wv_w,
        wq_b.reshape(1, D), wk_b.reshape(1, D), wv_b.reshape(1, D))

    qe, ke, ve = _run_gather_qkv(q, k, v, rowp, colp)
    ex128, msg_lo, msg_hi = _run_edge_logits(
        qe, ke, ve, relp3, spp3, rel_table, spt128)
    d0, d1, g0l, g1l, g0h, g1h = _run_agg3(ex128, msg_lo, msg_hi, colp)
    rw = _run_recip(d0, d1)
    recip_e = _run_recip_gather(rw, colp)
    attn_pad = _run_attn(ex128, recip_e)

    out2p = _run_final(
        xp, g0l, g1l, g0h, g1h, rw, dense_w, dense_b.reshape(1, D),
        ln2_w.reshape(1, D), ln2_b.reshape(1, D),
        ffn1_w, ffn1_b.reshape(1, DFF), ffn2_w, ffn2_b.reshape(1, D))

    return (out2p[:N], attn_pad[:E])


# speedup vs baseline: 1.0932x; 1.0608x over previous
import functools
import math

import jax
import jax.numpy as jnp
from jax import lax
from jax.experimental import pallas as pl
from jax.experimental.pallas import tpu as pltpu
from jax.experimental.pallas import tpu_sc as plsc

N = 10000
E = 160000
D = 256
H = 8
DH = 32
DFF = 1024
NUM_REL = 16
NUM_SP = 512

NP = 10240
EP = 163840
NWORK = 32
PER_W = EP // NWORK
EPH = EP // 2
PER_WH = EPH // NWORK

f32 = jnp.float32
i32 = jnp.int32


def _sds(shape, dtype):
    return jax.ShapeDtypeStruct(shape, dtype)


_BLKA = 256


def _qkv_body(x_ref, lnw_ref, lnb_ref, wq_ref, wk_ref, wv_ref,
              bq_ref, bk_ref, bv_ref, q_ref, k_ref, v_ref):
    x = x_ref[...]
    m = jnp.mean(x, axis=-1, keepdims=True)
    var = jnp.mean((x - m) * (x - m), axis=-1, keepdims=True)
    xn = (x - m) / jnp.sqrt(var + 1e-6) * lnw_ref[...] + lnb_ref[...]
    dn = (((1,), (1,)), ((), ()))
    q_ref[...] = lax.dot_general(xn, wq_ref[...], dn,
                                 preferred_element_type=f32) + bq_ref[...]
    k_ref[...] = lax.dot_general(xn, wk_ref[...], dn,
                                 preferred_element_type=f32) + bk_ref[...]
    v_ref[...] = lax.dot_general(xn, wv_ref[...], dn,
                                 preferred_element_type=f32) + bv_ref[...]


def _run_qkv(xp, lnw, lnb, wq, wk, wv, bq, bk, bv):
    full = pl.BlockSpec((D, D), lambda i: (0, 0))
    vec = pl.BlockSpec((1, D), lambda i: (0, 0))
    blk = pl.BlockSpec((_BLKA, D), lambda i: (i, 0))
    return pl.pallas_call(
        _qkv_body,
        grid=(NP // _BLKA,),
        in_specs=[blk, vec, vec, full, full, full, vec, vec, vec],
        out_specs=[blk, blk, blk],
        out_shape=[_sds((NP, D), f32)] * 3,
    )(xp, lnw, lnb, wq, wk, wv, bq, bk, bv)


_GC = 64


def _sc_gather_qkv_body(q_hbm, k_hbm, v_hbm, rowp_hbm, colp_hbm,
                        qe_hbm, ke_hbm, ve_hbm,
                        ra, ca, rb, cb, qa, ka, va, qb, kb, vb,
                        si, sg, so):
    c = lax.axis_index("c")
    s = lax.axis_index("s")
    base0 = (s * 2 + c) * PER_WH
    nit = PER_WH // (2 * _GC)

    def body(j, carry):
        ba = base0 + j * 2 * _GC
        bb = ba + _GC
        i0 = pltpu.async_copy(rowp_hbm.at[pl.ds(ba, _GC)], ra, si)
        i1 = pltpu.async_copy(colp_hbm.at[pl.ds(ba, _GC)], ca, si)
        i2 = pltpu.async_copy(rowp_hbm.at[pl.ds(bb, _GC)], rb, si)
        i3 = pltpu.async_copy(colp_hbm.at[pl.ds(bb, _GC)], cb, si)
        i0.wait()
        i1.wait()
        i2.wait()
        i3.wait()
        g0 = pltpu.async_copy(q_hbm.at[ca], qa, sg)
        g1 = pltpu.async_copy(k_hbm.at[ra], ka, sg)
        g2 = pltpu.async_copy(v_hbm.at[ra], va, sg)
        g3 = pltpu.async_copy(q_hbm.at[cb], qb, sg)
        g4 = pltpu.async_copy(k_hbm.at[rb], kb, sg)
        g5 = pltpu.async_copy(v_hbm.at[rb], vb, sg)
        g0.wait()
        g1.wait()
        g2.wait()
        g3.wait()
        g4.wait()
        g5.wait()
        o0 = pltpu.async_copy(qa, qe_hbm.at[pl.ds(ba, _GC)], so)
        o1 = pltpu.async_copy(ka, ke_hbm.at[pl.ds(ba, _GC)], so)
        o2 = pltpu.async_copy(va, ve_hbm.at[pl.ds(ba, _GC)], so)
        o3 = pltpu.async_copy(qb, qe_hbm.at[pl.ds(bb, _GC)], so)
        o4 = pltpu.async_copy(kb, ke_hbm.at[pl.ds(bb, _GC)], so)
        o5 = pltpu.async_copy(vb, ve_hbm.at[pl.ds(bb, _GC)], so)
        o0.wait()
        o1.wait()
        o2.wait()
        o3.wait()
        o4.wait()
        o5.wait()
        return carry

    lax.fori_loop(0, nit, body, 0)


def _run_gather_qkv(q, k, v, rowp, colp):
    mesh = plsc.VectorSubcoreMesh(core_axis_name="c", subcore_axis_name="s")
    fn = functools.partial(
        pl.kernel,
        mesh=mesh,
        out_type=[_sds((EPH, D), f32)] * 3,
        scratch_types=[
            pltpu.VMEM((_GC,), i32),
            pltpu.VMEM((_GC,), i32),
            pltpu.VMEM((_GC,), i32),
            pltpu.VMEM((_GC,), i32),
            pltpu.VMEM((_GC, D), f32),
            pltpu.VMEM((_GC, D), f32),
            pltpu.VMEM((_GC, D), f32),
            pltpu.VMEM((_GC, D), f32),
            pltpu.VMEM((_GC, D), f32),
            pltpu.VMEM((_GC, D), f32),
            pltpu.SemaphoreType.DMA,
            pltpu.SemaphoreType.DMA,
            pltpu.SemaphoreType.DMA,
        ],
    )(_sc_gather_qkv_body)
    return fn(q, k, v, rowp, colp)


_BLKE = 1024


def _edge_logits_body(qe_ref, ke_ref, ve_ref, rel_ref, sp_ref, relt_ref,
                      spt_ref, ex_ref, mlo_ref, mhi_ref):
    rel_ids = rel_ref[0]
    oh_rel = (lax.broadcasted_iota(i32, (NUM_REL, _BLKE), 0)
              == rel_ids).astype(f32)
    dn0 = (((0,), (0,)), ((), ()))
    rel_e = lax.dot_general(oh_rel, relt_ref[...], dn0,
                            preferred_element_type=f32)
    qe = qe_ref[...] + rel_e
    ke = ke_ref[...] + rel_e
    prod = qe * ke
    s128 = (lax.broadcasted_iota(i32, (D, 128), 0) // DH
            == lax.broadcasted_iota(i32, (D, 128), 1)).astype(f32)
    dn1 = (((1,), (0,)), ((), ()))
    logits = lax.dot_general(prod, s128, dn1,
                             preferred_element_type=f32) / math.sqrt(DH)
    sp_ids = sp_ref[0]
    oh_sp = (lax.broadcasted_iota(i32, (NUM_SP, _BLKE), 0)
             == sp_ids).astype(f32)
    sp_e = lax.dot_general(oh_sp, spt_ref[...], dn0,
                           preferred_element_type=f32)
    head_mask = lax.broadcasted_iota(i32, (_BLKE, 128), 1) < H
    ex = jnp.where(head_mask, jnp.exp(logits + sp_e), 0.0)
    ex_ref[...] = ex
    r = (lax.broadcasted_iota(i32, (128, D), 0)
         == lax.broadcasted_iota(i32, (128, D), 1) // DH).astype(f32)
    ex256 = lax.dot_general(ex, r, dn1, preferred_element_type=f32)
    msg = ve_ref[...] * ex256
    mlo_ref[...] = msg[:, :128]
    mhi_ref[...] = msg[:, 128:]


def _run_edge_logits(qe, ke, ve, relp3, spp3, relt, spt128):
    eblk = pl.BlockSpec((_BLKE, D), lambda i: (i, 0))
    hblk = pl.BlockSpec((_BLKE, 128), lambda i: (i, 0))
    iblk = pl.BlockSpec((1, 1, _BLKE), lambda i: (i, 0, 0))
    return pl.pallas_call(
        _edge_logits_body,
        grid=(EPH // _BLKE,),
        in_specs=[
            eblk, eblk, eblk, iblk, iblk,
            pl.BlockSpec((NUM_REL, D), lambda i: (0, 0)),
            pl.BlockSpec((NUM_SP, 128), lambda i: (0, 0)),
        ],
        out_specs=[hblk, hblk, hblk],
        out_shape=[_sds((EPH, 128), f32)] * 3,
    )(qe, ke, ve, relp3, spp3, relt, spt128)


_BLKR = 512
_ZR = 64


def _recip_body(d0_ref, d1_ref, rw_ref):
    den = d0_ref[...] + d1_ref[...]
    lane = lax.broadcasted_iota(i32, (_BLKR, 128), 1)
    rw_ref[...] = jnp.where(lane < H, 1.0 / (den + 1e-16), 0.0)


def _run_recip(d0, d1):
    dblk = pl.BlockSpec((_BLKR, 128), lambda i: (i, 0))
    return pl.pallas_call(
        _recip_body,
        grid=(NP // _BLKR,),
        in_specs=[dblk, dblk],
        out_specs=dblk,
        out_shape=_sds((NP, 128), f32),
    )(d0, d1)


_EC = 128


def _sc_recip_gather_body(rw_hbm, colp_hbm, re_hbm,
                          c0, c1, c2, c3, b0, b1, b2, b3, si, sg, so):
    c = lax.axis_index("c")
    s = lax.axis_index("s")
    base0 = (s * 2 + c) * PER_WH
    nit = PER_WH // (4 * _EC)
    cbs = (c0, c1, c2, c3)
    bbs = (b0, b1, b2, b3)

    def body(j, carry):
        b = base0 + j * 4 * _EC
        iws = [pltpu.async_copy(colp_hbm.at[pl.ds(b + t * _EC, _EC)], cbs[t],
                                si) for t in range(4)]
        for w in iws:
            w.wait()
        gws = [pltpu.async_copy(rw_hbm.at[cbs[t]], bbs[t], sg)
               for t in range(4)]
        for w in gws:
            w.wait()
        ows = [pltpu.async_copy(bbs[t], re_hbm.at[pl.ds(b + t * _EC, _EC)],
                                so) for t in range(4)]
        for w in ows:
            w.wait()
        return carry

    lax.fori_loop(0, nit, body, 0)


def _run_recip_gather(rw, colp):
    mesh = plsc.VectorSubcoreMesh(core_axis_name="c", subcore_axis_name="s")
    fn = functools.partial(
        pl.kernel,
        mesh=mesh,
        out_type=_sds((EPH, 128), f32),
        scratch_types=(
            [pltpu.VMEM((_EC,), i32)] * 4
            + [pltpu.VMEM((_EC, 128), f32)] * 4
            + [pltpu.SemaphoreType.DMA] * 3
        ),
    )(_sc_recip_gather_body)
    return fn(rw, colp)


def _attn_body(ex_ref, re_ref, attn_ref):
    attn_ref[...] = (ex_ref[...] * re_ref[...])[:, :H]


def _run_attn(ex128, recip_e):
    rblk = pl.BlockSpec((_BLKE, 128), lambda i: (i, 0))
    return pl.pallas_call(
        _attn_body,
        grid=(EPH // _BLKE,),
        in_specs=[rblk, rblk],
        out_specs=pl.BlockSpec((_BLKE, H), lambda i: (i, 0)),
        out_shape=_sds((EPH, H), f32),
    )(ex128, recip_e)


_MC = 128


def _sc_agg_body(ex_a, ex_b, mlo_a, mlo_b, mhi_a, mhi_b, col_a, col_b,
                 d0_hbm, d1_hbm, g0l_hbm, g1l_hbm, g0h_hbm, g1h_hbm,
                 acc, zb, ma, mb, ca, cb, si, sd):
    c = lax.axis_index("c")
    s = lax.axis_index("s")
    wid = s * 2 + c

    def zbody(t, carry):
        i = t // 8
        jj = (t % 8) * 16
        zb[i, pl.ds(jj, 16)] = jnp.zeros((16,), f32)
        return carry

    lax.fori_loop(0, _ZR * 8, zbody, 0)
    rows_per_tile = NP // 16
    base0 = wid * PER_WH

    def half_scan(src_hbm, colp_hbm):
        def body(j, carry):
            ba = base0 + j * 2 * _MC
            bb = ba + _MC
            i0 = pltpu.async_copy(colp_hbm.at[pl.ds(ba, _MC)], ca, si)
            i1 = pltpu.async_copy(colp_hbm.at[pl.ds(bb, _MC)], cb, si)
            d0 = pltpu.async_copy(src_hbm.at[pl.ds(ba, _MC)], ma, sd)
            d1 = pltpu.async_copy(src_hbm.at[pl.ds(bb, _MC)], mb, sd)
            i0.wait()
            i1.wait()
            d0.wait()
            d1.wait()
            pltpu.sync_copy(ma, acc.at[ca], add=True)
            pltpu.sync_copy(mb, acc.at[cb], add=True)
            return carry

        lax.fori_loop(0, PER_WH // (2 * _MC), body, 0)

    def one_pass(src_a, src_b, o0_hbm, o1_hbm):
        def zcbody(b, carry):
            pltpu.sync_copy(
                zb, acc.at[pl.ds(s * rows_per_tile + b * _ZR, _ZR)])
            return carry

        lax.fori_loop(0, rows_per_tile // _ZR, zcbody, 0)
        plsc.subcore_barrier()
        half_scan(src_a, col_a)
        half_scan(src_b, col_b)
        plsc.subcore_barrier()

        @pl.when(jnp.logical_and(s == 0, c == 0))
        def _():
            pltpu.sync_copy(acc, o0_hbm)

        @pl.when(jnp.logical_and(s == 0, c == 1))
        def _():
            pltpu.sync_copy(acc, o1_hbm)

        plsc.subcore_barrier()

    one_pass(ex_a, ex_b, d0_hbm, d1_hbm)
    one_pass(mlo_a, mlo_b, g0l_hbm, g1l_hbm)
    one_pass(mhi_a, mhi_b, g0h_hbm, g1h_hbm)


def _run_agg3(ex_a, ex_b, mlo_a, mlo_b, mhi_a, mhi_b, col_a, col_b):
    mesh = plsc.VectorSubcoreMesh(core_axis_name="c", subcore_axis_name="s")
    fn = functools.partial(
        pl.kernel,
        mesh=mesh,
        out_type=[_sds((NP, 128), f32)] * 6,
        scratch_types=[
            pltpu.VMEM_SHARED((NP, 128), f32),
            pltpu.VMEM((_ZR, 128), f32),
            pltpu.VMEM((_MC, 128), f32),
            pltpu.VMEM((_MC, 128), f32),
            pltpu.VMEM((_MC,), i32),
            pltpu.VMEM((_MC,), i32),
            pltpu.SemaphoreType.DMA,
            pltpu.SemaphoreType.DMA,
        ],
    )(_sc_agg_body)
    return fn(ex_a, ex_b, mlo_a, mlo_b, mhi_a, mhi_b, col_a, col_b)


_BLKH = 256


def _final_body(xp_ref, g0l_ref, g1l_ref, g0h_ref, g1h_ref, rw_ref,
                dw_ref, db_ref, ln2w_ref, ln2b_ref,
                f1w_ref, f1b_ref, f2w_ref, f2b_ref, out_ref):
    agg_un = jnp.concatenate(
        [g0l_ref[...] + g1l_ref[...], g0h_ref[...] + g1h_ref[...]], axis=-1)
    r = (lax.broadcasted_iota(i32, (128, D), 0)
         == lax.broadcasted_iota(i32, (128, D), 1) // DH).astype(f32)
    rdn = (((1,), (0,)), ((), ()))
    rec256 = lax.dot_general(rw_ref[...], r, rdn, preferred_element_type=f32)
    agg = agg_un * rec256
    dn = (((1,), (1,)), ((), ()))
    attn_out = lax.dot_general(agg, dw_ref[...], dn,
                               preferred_element_type=f32) + db_ref[...]
    out1 = attn_out + xp_ref[...]
    m = jnp.mean(out1, axis=-1, keepdims=True)
    var = jnp.mean((out1 - m) * (out1 - m), axis=-1, keepdims=True)
    o1n = (out1 - m) / jnp.sqrt(var + 1e-6) * ln2w_ref[...] + ln2b_ref[...]
    h1 = jnp.maximum(
        lax.dot_general(o1n, f1w_ref[...], dn,
                        preferred_element_type=f32) + f1b_ref[...], 0.0)
    ffn = lax.dot_general(h1, f2w_ref[...], dn,
                          preferred_element_type=f32) + f2b_ref[...]
    out_ref[...] = out1 + ffn


def _run_final(xp, g0l, g1l, g0h, g1h, rw, dw, db, ln2w, ln2b,
               f1w, f1b, f2w, f2b):
    nblk = pl.BlockSpec((_BLKH, D), lambda i: (i, 0))
    hblk = pl.BlockSpec((_BLKH, 128), lambda i: (i, 0))
    vec = pl.BlockSpec((1, D), lambda i: (0, 0))
    return pl.pallas_call(
        _final_body,
        grid=(NP // _BLKH,),
        in_specs=[
            nblk, hblk, hblk, hblk, hblk, hblk,
            pl.BlockSpec((D, D), lambda i: (0, 0)), vec, vec, vec,
            pl.BlockSpec((DFF, D), lambda i: (0, 0)),
            pl.BlockSpec((1, DFF), lambda i: (0, 0)),
            pl.BlockSpec((D, DFF), lambda i: (0, 0)), vec,
        ],
        out_specs=nblk,
        out_shape=_sds((NP, D), f32),
    )(xp, g0l, g1l, g0h, g1h, rw, dw, db, ln2w, ln2b, f1w, f1b, f2w, f2b)


def kernel(feature, sp_edge_index, sp_value, edge_rel, ln1_w, ln1_b,
           wq_w, wq_b, wk_w, wk_b, wv_w, wv_b, rel_table, spatial_table,
           dense_w, dense_b, ln2_w, ln2_b, ffn1_w, ffn1_b, ffn2_w, ffn2_b):
    xp = jnp.pad(feature, ((0, NP - N), (0, 0)))
    rowp = jnp.concatenate(
        [sp_edge_index[0], jnp.full((EP - E,), N, i32)])
    colp = jnp.concatenate(
        [sp_edge_index[1], jnp.full((EP - E,), N, i32)])
    relp3 = jnp.concatenate(
        [edge_rel, jnp.zeros((EP - E,), i32)]).reshape(EP // _BLKE, 1, _BLKE)
    spp3 = jnp.concatenate(
        [sp_value, jnp.zeros((EP - E,), i32)]).reshape(EP // _BLKE, 1, _BLKE)
    spt128 = jnp.pad(spatial_table, ((0, 0), (0, 128 - H)))

    q, k, v = _run_qkv(
        xp, ln1_w.reshape(1, D), ln1_b.reshape(1, D),
        wq_w, wk_w, wv_w,
        wq_b.reshape(1, D), wk_b.reshape(1, D), wv_b.reshape(1, D))

    row_a, row_b = rowp[:EPH], rowp[EPH:]
    col_a, col_b = colp[:EPH], colp[EPH:]
    nb3 = EPH // _BLKE
    qe_a, ke_a, ve_a = _run_gather_qkv(q, k, v, row_a, col_a)
    ex_a, mlo_a, mhi_a = _run_edge_logits(
        qe_a, ke_a, ve_a, relp3[:nb3], spp3[:nb3], rel_table, spt128)
    qe_b, ke_b, ve_b = _run_gather_qkv(q, k, v, row_b, col_b)
    ex_b, mlo_b, mhi_b = _run_edge_logits(
        qe_b, ke_b, ve_b, relp3[nb3:], spp3[nb3:], rel_table, spt128)
    d0, d1, g0l, g1l, g0h, g1h = _run_agg3(
        ex_a, ex_b, mlo_a, mlo_b, mhi_a, mhi_b, col_a, col_b)
    rw = _run_recip(d0, d1)
    re_a = _run_recip_gather(rw, col_a)
    attn_a = _run_attn(ex_a, re_a)
    re_b = _run_recip_gather(rw, col_b)
    attn_b = _run_attn(ex_b, re_b)
    attn_pad = jnp.concatenate([attn_a, attn_b])

    out2p = _run_final(
        xp, g0l, g1l, g0h, g1h, rw, dense_w, dense_b.reshape(1, D),
        ln2_w.reshape(1, D), ln2_b.reshape(1, D),
        ffn1_w, ffn1_b.reshape(1, DFF), ffn2_w, ffn2_b.reshape(1, D))

    return (out2p[:N], attn_pad[:E])


# speedup vs baseline: 1.1835x; 1.0826x over previous
import functools
import math

import jax
import jax.numpy as jnp
from jax import lax
from jax.experimental import pallas as pl
from jax.experimental.pallas import tpu as pltpu
from jax.experimental.pallas import tpu_sc as plsc

N = 10000
E = 160000
D = 256
H = 8
DH = 32
DFF = 1024
NUM_REL = 16
NUM_SP = 512

NP = 10240
EP = 163840
NWORK = 32
PER_W = EP // NWORK
NSPLIT = 4
EPH = EP // NSPLIT
PER_WH = EPH // NWORK

f32 = jnp.float32
i32 = jnp.int32


def _sds(shape, dtype):
    return jax.ShapeDtypeStruct(shape, dtype)


_BLKA = 256


def _qkv_body(x_ref, lnw_ref, lnb_ref, wq_ref, wk_ref, wv_ref,
              bq_ref, bk_ref, bv_ref, q_ref, k_ref, v_ref):
    x = x_ref[...]
    m = jnp.mean(x, axis=-1, keepdims=True)
    var = jnp.mean((x - m) * (x - m), axis=-1, keepdims=True)
    xn = (x - m) / jnp.sqrt(var + 1e-6) * lnw_ref[...] + lnb_ref[...]
    dn = (((1,), (1,)), ((), ()))
    q_ref[...] = lax.dot_general(xn, wq_ref[...], dn,
                                 preferred_element_type=f32) + bq_ref[...]
    k_ref[...] = lax.dot_general(xn, wk_ref[...], dn,
                                 preferred_element_type=f32) + bk_ref[...]
    v_ref[...] = lax.dot_general(xn, wv_ref[...], dn,
                                 preferred_element_type=f32) + bv_ref[...]


def _run_qkv(xp, lnw, lnb, wq, wk, wv, bq, bk, bv):
    full = pl.BlockSpec((D, D), lambda i: (0, 0))
    vec = pl.BlockSpec((1, D), lambda i: (0, 0))
    blk = pl.BlockSpec((_BLKA, D), lambda i: (i, 0))
    return pl.pallas_call(
        _qkv_body,
        grid=(NP // _BLKA,),
        in_specs=[blk, vec, vec, full, full, full, vec, vec, vec],
        out_specs=[blk, blk, blk],
        out_shape=[_sds((NP, D), f32)] * 3,
    )(xp, lnw, lnb, wq, wk, wv, bq, bk, bv)


_GC = 64


def _sc_gather_qkv_body(q_hbm, k_hbm, v_hbm, rowp_hbm, colp_hbm,
                        qe_hbm, ke_hbm, ve_hbm,
                        ra, ca, rb, cb, qa, ka, va, qb, kb, vb,
                        si, sg, so):
    c = lax.axis_index("c")
    s = lax.axis_index("s")
    base0 = (s * 2 + c) * PER_WH
    nit = PER_WH // (2 * _GC)

    def body(j, carry):
        ba = base0 + j * 2 * _GC
        bb = ba + _GC
        i0 = pltpu.async_copy(rowp_hbm.at[pl.ds(ba, _GC)], ra, si)
        i1 = pltpu.async_copy(colp_hbm.at[pl.ds(ba, _GC)], ca, si)
        i2 = pltpu.async_copy(rowp_hbm.at[pl.ds(bb, _GC)], rb, si)
        i3 = pltpu.async_copy(colp_hbm.at[pl.ds(bb, _GC)], cb, si)
        i0.wait()
        i1.wait()
        i2.wait()
        i3.wait()
        g0 = pltpu.async_copy(q_hbm.at[ca], qa, sg)
        g1 = pltpu.async_copy(k_hbm.at[ra], ka, sg)
        g2 = pltpu.async_copy(v_hbm.at[ra], va, sg)
        g3 = pltpu.async_copy(q_hbm.at[cb], qb, sg)
        g4 = pltpu.async_copy(k_hbm.at[rb], kb, sg)
        g5 = pltpu.async_copy(v_hbm.at[rb], vb, sg)
        g0.wait()
        g1.wait()
        g2.wait()
        g3.wait()
        g4.wait()
        g5.wait()
        o0 = pltpu.async_copy(qa, qe_hbm.at[pl.ds(ba, _GC)], so)
        o1 = pltpu.async_copy(ka, ke_hbm.at[pl.ds(ba, _GC)], so)
        o2 = pltpu.async_copy(va, ve_hbm.at[pl.ds(ba, _GC)], so)
        o3 = pltpu.async_copy(qb, qe_hbm.at[pl.ds(bb, _GC)], so)
        o4 = pltpu.async_copy(kb, ke_hbm.at[pl.ds(bb, _GC)], so)
        o5 = pltpu.async_copy(vb, ve_hbm.at[pl.ds(bb, _GC)], so)
        o0.wait()
        o1.wait()
        o2.wait()
        o3.wait()
        o4.wait()
        o5.wait()
        return carry

    lax.fori_loop(0, nit, body, 0)


def _run_gather_qkv(q, k, v, rowp, colp):
    mesh = plsc.VectorSubcoreMesh(core_axis_name="c", subcore_axis_name="s")
    fn = functools.partial(
        pl.kernel,
        mesh=mesh,
        out_type=[_sds((EPH, D), f32)] * 3,
        scratch_types=[
            pltpu.VMEM((_GC,), i32),
            pltpu.VMEM((_GC,), i32),
            pltpu.VMEM((_GC,), i32),
            pltpu.VMEM((_GC,), i32),
            pltpu.VMEM((_GC, D), f32),
            pltpu.VMEM((_GC, D), f32),
            pltpu.VMEM((_GC, D), f32),
            pltpu.VMEM((_GC, D), f32),
            pltpu.VMEM((_GC, D), f32),
            pltpu.VMEM((_GC, D), f32),
            pltpu.SemaphoreType.DMA,
            pltpu.SemaphoreType.DMA,
            pltpu.SemaphoreType.DMA,
        ],
    )(_sc_gather_qkv_body)
    return fn(q, k, v, rowp, colp)


_BLKE = 1024


def _edge_logits_body(qe_ref, ke_ref, ve_ref, rel_ref, sp_ref, relt_ref,
                      spt_ref, ex_ref, mlo_ref, mhi_ref):
    rel_ids = rel_ref[0]
    oh_rel = (lax.broadcasted_iota(i32, (NUM_REL, _BLKE), 0)
              == rel_ids).astype(f32)
    dn0 = (((0,), (0,)), ((), ()))
    rel_e = lax.dot_general(oh_rel, relt_ref[...], dn0,
                            preferred_element_type=f32)
    qe = qe_ref[...] + rel_e
    ke = ke_ref[...] + rel_e
    prod = qe * ke
    s128 = (lax.broadcasted_iota(i32, (D, 128), 0) // DH
            == lax.broadcasted_iota(i32, (D, 128), 1)).astype(f32)
    dn1 = (((1,), (0,)), ((), ()))
    logits = lax.dot_general(prod, s128, dn1,
                             preferred_element_type=f32) / math.sqrt(DH)
    sp_ids = sp_ref[0]
    oh_sp = (lax.broadcasted_iota(i32, (NUM_SP, _BLKE), 0)
             == sp_ids).astype(f32)
    sp_e = lax.dot_general(oh_sp, spt_ref[...], dn0,
                           preferred_element_type=f32)
    head_mask = lax.broadcasted_iota(i32, (_BLKE, 128), 1) < H
    ex = jnp.where(head_mask, jnp.exp(logits + sp_e), 0.0)
    ex_ref[...] = ex
    r = (lax.broadcasted_iota(i32, (128, D), 0)
         == lax.broadcasted_iota(i32, (128, D), 1) // DH).astype(f32)
    ex256 = lax.dot_general(ex, r, dn1, preferred_element_type=f32)
    msg = ve_ref[...] * ex256
    mlo_ref[...] = msg[:, :128]
    mhi_ref[...] = msg[:, 128:]


def _run_edge_logits(qe, ke, ve, relp3, spp3, relt, spt128):
    eblk = pl.BlockSpec((_BLKE, D), lambda i: (i, 0))
    hblk = pl.BlockSpec((_BLKE, 128), lambda i: (i, 0))
    iblk = pl.BlockSpec((1, 1, _BLKE), lambda i: (i, 0, 0))
    return pl.pallas_call(
        _edge_logits_body,
        grid=(EPH // _BLKE,),
        in_specs=[
            eblk, eblk, eblk, iblk, iblk,
            pl.BlockSpec((NUM_REL, D), lambda i: (0, 0)),
            pl.BlockSpec((NUM_SP, 128), lambda i: (0, 0)),
        ],
        out_specs=[hblk, hblk, hblk],
        out_shape=[_sds((EPH, 128), f32)] * 3,
    )(qe, ke, ve, relp3, spp3, relt, spt128)


_BLKR = 512
_ZR = 64


def _recip_body(d0_ref, d1_ref, rw_ref):
    den = d0_ref[...] + d1_ref[...]
    lane = lax.broadcasted_iota(i32, (_BLKR, 128), 1)
    rw_ref[...] = jnp.where(lane < H, 1.0 / (den + 1e-16), 0.0)


def _run_recip(d0, d1):
    dblk = pl.BlockSpec((_BLKR, 128), lambda i: (i, 0))
    return pl.pallas_call(
        _recip_body,
        grid=(NP // _BLKR,),
        in_specs=[dblk, dblk],
        out_specs=dblk,
        out_shape=_sds((NP, 128), f32),
    )(d0, d1)


_EC = 128


def _sc_recip_gather_body(rw_hbm, colp_hbm, re_hbm,
                          c0, c1, c2, c3, b0, b1, b2, b3, si, sg, so):
    c = lax.axis_index("c")
    s = lax.axis_index("s")
    base0 = (s * 2 + c) * PER_WH
    nit = PER_WH // (4 * _EC)
    cbs = (c0, c1, c2, c3)
    bbs = (b0, b1, b2, b3)

    def body(j, carry):
        b = base0 + j * 4 * _EC
        iws = [pltpu.async_copy(colp_hbm.at[pl.ds(b + t * _EC, _EC)], cbs[t],
                                si) for t in range(4)]
        for w in iws:
            w.wait()
        gws = [pltpu.async_copy(rw_hbm.at[cbs[t]], bbs[t], sg)
               for t in range(4)]
        for w in gws:
            w.wait()
        ows = [pltpu.async_copy(bbs[t], re_hbm.at[pl.ds(b + t * _EC, _EC)],
                                so) for t in range(4)]
        for w in ows:
            w.wait()
        return carry

    lax.fori_loop(0, nit, body, 0)


def _run_recip_gather(rw, colp):
    mesh = plsc.VectorSubcoreMesh(core_axis_name="c", subcore_axis_name="s")
    fn = functools.partial(
        pl.kernel,
        mesh=mesh,
        out_type=_sds((EPH, 128), f32),
        scratch_types=(
            [pltpu.VMEM((_EC,), i32)] * 4
            + [pltpu.VMEM((_EC, 128), f32)] * 4
            + [pltpu.SemaphoreType.DMA] * 3
        ),
    )(_sc_recip_gather_body)
    return fn(rw, colp)


def _attn_body(ex_ref, re_ref, attn_ref):
    attn_ref[...] = (ex_ref[...] * re_ref[...])[:, :H]


def _run_attn(ex128, recip_e):
    rblk = pl.BlockSpec((_BLKE, 128), lambda i: (i, 0))
    return pl.pallas_call(
        _attn_body,
        grid=(EPH // _BLKE,),
        in_specs=[rblk, rblk],
        out_specs=pl.BlockSpec((_BLKE, H), lambda i: (i, 0)),
        out_shape=_sds((EPH, H), f32),
    )(ex128, recip_e)


_MC = 128


def _sc_agg_body(*refs):
    (ex_s, mlo_s, mhi_s, col_s) = (refs[0:NSPLIT], refs[NSPLIT:2 * NSPLIT],
                                   refs[2 * NSPLIT:3 * NSPLIT],
                                   refs[3 * NSPLIT:4 * NSPLIT])
    (d0_hbm, d1_hbm, g0l_hbm, g1l_hbm, g0h_hbm, g1h_hbm,
     acc, zb, ma, mb, ca, cb, si, sd) = refs[4 * NSPLIT:]
    c = lax.axis_index("c")
    s = lax.axis_index("s")
    wid = s * 2 + c

    def zbody(t, carry):
        i = t // 8
        jj = (t % 8) * 16
        zb[i, pl.ds(jj, 16)] = jnp.zeros((16,), f32)
        return carry

    lax.fori_loop(0, _ZR * 8, zbody, 0)
    rows_per_tile = NP // 16
    base0 = wid * PER_WH

    def half_scan(src_hbm, colp_hbm):
        def body(j, carry):
            ba = base0 + j * 2 * _MC
            bb = ba + _MC
            i0 = pltpu.async_copy(colp_hbm.at[pl.ds(ba, _MC)], ca, si)
            i1 = pltpu.async_copy(colp_hbm.at[pl.ds(bb, _MC)], cb, si)
            d0 = pltpu.async_copy(src_hbm.at[pl.ds(ba, _MC)], ma, sd)
            d1 = pltpu.async_copy(src_hbm.at[pl.ds(bb, _MC)], mb, sd)
            i0.wait()
            i1.wait()
            d0.wait()
            d1.wait()
            pltpu.sync_copy(ma, acc.at[ca], add=True)
            pltpu.sync_copy(mb, acc.at[cb], add=True)
            return carry

        lax.fori_loop(0, PER_WH // (2 * _MC), body, 0)

    def one_pass(srcs, o0_hbm, o1_hbm):
        def zcbody(b, carry):
            pltpu.sync_copy(
                zb, acc.at[pl.ds(s * rows_per_tile + b * _ZR, _ZR)])
            return carry

        lax.fori_loop(0, rows_per_tile // _ZR, zcbody, 0)
        plsc.subcore_barrier()
        for t in range(NSPLIT):
            half_scan(srcs[t], col_s[t])
        plsc.subcore_barrier()

        @pl.when(jnp.logical_and(s == 0, c == 0))
        def _():
            pltpu.sync_copy(acc, o0_hbm)

        @pl.when(jnp.logical_and(s == 0, c == 1))
        def _():
            pltpu.sync_copy(acc, o1_hbm)

        plsc.subcore_barrier()

    one_pass(ex_s, d0_hbm, d1_hbm)
    one_pass(mlo_s, g0l_hbm, g1l_hbm)
    one_pass(mhi_s, g0h_hbm, g1h_hbm)


def _run_agg3(ex_s, mlo_s, mhi_s, col_s):
    mesh = plsc.VectorSubcoreMesh(core_axis_name="c", subcore_axis_name="s")
    fn = functools.partial(
        pl.kernel,
        mesh=mesh,
        out_type=[_sds((NP, 128), f32)] * 6,
        scratch_types=[
            pltpu.VMEM_SHARED((NP, 128), f32),
            pltpu.VMEM((_ZR, 128), f32),
            pltpu.VMEM((_MC, 128), f32),
            pltpu.VMEM((_MC, 128), f32),
            pltpu.VMEM((_MC,), i32),
            pltpu.VMEM((_MC,), i32),
            pltpu.SemaphoreType.DMA,
            pltpu.SemaphoreType.DMA,
        ],
    )(_sc_agg_body)
    return fn(*ex_s, *mlo_s, *mhi_s, *col_s)


_BLKH = 256


def _final_body(xp_ref, g0l_ref, g1l_ref, g0h_ref, g1h_ref, rw_ref,
                dw_ref, db_ref, ln2w_ref, ln2b_ref,
                f1w_ref, f1b_ref, f2w_ref, f2b_ref, out_ref):
    agg_un = jnp.concatenate(
        [g0l_ref[...] + g1l_ref[...], g0h_ref[...] + g1h_ref[...]], axis=-1)
    r = (lax.broadcasted_iota(i32, (128, D), 0)
         == lax.broadcasted_iota(i32, (128, D), 1) // DH).astype(f32)
    rdn = (((1,), (0,)), ((), ()))
    rec256 = lax.dot_general(rw_ref[...], r, rdn, preferred_element_type=f32)
    agg = agg_un * rec256
    dn = (((1,), (1,)), ((), ()))
    attn_out = lax.dot_general(agg, dw_ref[...], dn,
                               preferred_element_type=f32) + db_ref[...]
    out1 = attn_out + xp_ref[...]
    m = jnp.mean(out1, axis=-1, keepdims=True)
    var = jnp.mean((out1 - m) * (out1 - m), axis=-1, keepdims=True)
    o1n = (out1 - m) / jnp.sqrt(var + 1e-6) * ln2w_ref[...] + ln2b_ref[...]
    h1 = jnp.maximum(
        lax.dot_general(o1n, f1w_ref[...], dn,
                        preferred_element_type=f32) + f1b_ref[...], 0.0)
    ffn = lax.dot_general(h1, f2w_ref[...], dn,
                          preferred_element_type=f32) + f2b_ref[...]
    out_ref[...] = out1 + ffn


def _run_final(xp, g0l, g1l, g0h, g1h, rw, dw, db, ln2w, ln2b,
               f1w, f1b, f2w, f2b):
    nblk = pl.BlockSpec((_BLKH, D), lambda i: (i, 0))
    hblk = pl.BlockSpec((_BLKH, 128), lambda i: (i, 0))
    vec = pl.BlockSpec((1, D), lambda i: (0, 0))
    return pl.pallas_call(
        _final_body,
        grid=(NP // _BLKH,),
        in_specs=[
            nblk, hblk, hblk, hblk, hblk, hblk,
            pl.BlockSpec((D, D), lambda i: (0, 0)), vec, vec, vec,
            pl.BlockSpec((DFF, D), lambda i: (0, 0)),
            pl.BlockSpec((1, DFF), lambda i: (0, 0)),
            pl.BlockSpec((D, DFF), lambda i: (0, 0)), vec,
        ],
        out_specs=nblk,
        out_shape=_sds((NP, D), f32),
    )(xp, g0l, g1l, g0h, g1h, rw, dw, db, ln2w, ln2b, f1w, f1b, f2w, f2b)


def kernel(feature, sp_edge_index, sp_value, edge_rel, ln1_w, ln1_b,
           wq_w, wq_b, wk_w, wk_b, wv_w, wv_b, rel_table, spatial_table,
           dense_w, dense_b, ln2_w, ln2_b, ffn1_w, ffn1_b, ffn2_w, ffn2_b):
    xp = jnp.pad(feature, ((0, NP - N), (0, 0)))
    rowp = jnp.concatenate(
        [sp_edge_index[0], jnp.full((EP - E,), N, i32)])
    colp = jnp.concatenate(
        [sp_edge_index[1], jnp.full((EP - E,), N, i32)])
    relp3 = jnp.concatenate(
        [edge_rel, jnp.zeros((EP - E,), i32)]).reshape(EP // _BLKE, 1, _BLKE)
    spp3 = jnp.concatenate(
        [sp_value, jnp.zeros((EP - E,), i32)]).reshape(EP // _BLKE, 1, _BLKE)
    spt128 = jnp.pad(spatial_table, ((0, 0), (0, 128 - H)))

    q, k, v = _run_qkv(
        xp, ln1_w.reshape(1, D), ln1_b.reshape(1, D),
        wq_w, wk_w, wv_w,
        wq_b.reshape(1, D), wk_b.reshape(1, D), wv_b.reshape(1, D))

    nb3 = EPH // _BLKE
    row_s = [rowp[t * EPH:(t + 1) * EPH] for t in range(NSPLIT)]
    col_s = [colp[t * EPH:(t + 1) * EPH] for t in range(NSPLIT)]
    ex_s, mlo_s, mhi_s = [], [], []
    for t in range(NSPLIT):
        qe_t, ke_t, ve_t = _run_gather_qkv(q, k, v, row_s[t], col_s[t])
        ex_t, mlo_t, mhi_t = _run_edge_logits(
            qe_t, ke_t, ve_t, relp3[t * nb3:(t + 1) * nb3],
            spp3[t * nb3:(t + 1) * nb3], rel_table, spt128)
        ex_s.append(ex_t)
        mlo_s.append(mlo_t)
        mhi_s.append(mhi_t)
    d0, d1, g0l, g1l, g0h, g1h = _run_agg3(ex_s, mlo_s, mhi_s, col_s)
    rw = _run_recip(d0, d1)
    attn_parts = []
    for t in range(NSPLIT):
        re_t = _run_recip_gather(rw, col_s[t])
        attn_parts.append(_run_attn(ex_s[t], re_t))
    attn_pad = jnp.concatenate(attn_parts)

    out2p = _run_final(
        xp, g0l, g1l, g0h, g1h, rw, dense_w, dense_b.reshape(1, D),
        ln2_w.reshape(1, D), ln2_b.reshape(1, D),
        ffn1_w, ffn1_b.reshape(1, DFF), ffn2_w, ffn2_b.reshape(1, D))

    return (out2p[:N], attn_pad[:E])
